# baseline (speedup 1.0000x reference)
import functools

import jax
import jax.numpy as jnp
from jax import lax
from jax.experimental import pallas as pl
from jax.experimental.pallas import tpu as pltpu
from jax.experimental.pallas import tpu_sc as plsc

_NC = 2
_NS = 16
_NW = _NC * _NS


def _proj_body(x_ref, w_ref, b_ref, o_ref):
    o_ref[...] = lax.dot_general(
        x_ref[...], w_ref[...],
        dimension_numbers=(((1,), (1,)), ((), ())),
        preferred_element_type=jnp.float32,
    ) + b_ref[...]


@functools.lru_cache(maxsize=None)
def _make_project(v, d, bm):
    grid = (pl.cdiv(v, bm),)
    return pl.pallas_call(
        _proj_body,
        grid=grid,
        in_specs=[
            pl.BlockSpec((bm, d), lambda i: (i, 0)),
            pl.BlockSpec((d, d), lambda i: (0, 0)),
            pl.BlockSpec((1, d), lambda i: (0, 0)),
        ],
        out_specs=pl.BlockSpec((bm, d), lambda i: (i, 0)),
        out_shape=jax.ShapeDtypeStruct((v, d), jnp.float32),
    )


@functools.lru_cache(maxsize=None)
def _make_gather(b_total, d, ch):
    b_per_w = b_total // _NW
    n_ch = b_per_w // ch
    mesh = plsc.VectorSubcoreMesh(core_axis_name="c", subcore_axis_name="s")

    @functools.partial(
        pl.kernel,
        mesh=mesh,
        out_type=jax.ShapeDtypeStruct((b_total, d), jnp.float32),
        scratch_types=[
            pltpu.VMEM((ch,), jnp.int32),
            pltpu.VMEM((ch, d), jnp.float32),
            pltpu.SemaphoreType.DMA,
        ],
    )
    def gather_kernel(table_hbm, idx_hbm, out_hbm, idx_v, rows_v, sem):
        wid = lax.axis_index("s") * _NC + lax.axis_index("c")
        base = wid * b_per_w

        def body(i, carry):
            off = base + i * ch
            pltpu.sync_copy(idx_hbm.at[pl.ds(off, ch)], idx_v)
            pltpu.async_copy(table_hbm.at[idx_v], rows_v, sem).wait()
            pltpu.sync_copy(rows_v, out_hbm.at[pl.ds(off, ch)])
            return carry

        lax.fori_loop(0, n_ch, body, 0)

    return gather_kernel


def kernel(input_ids, emb_table, W, b):
    v, d = emb_table.shape
    n_seq, seq = input_ids.shape
    seq_p = 24
    ids_p = jnp.pad(input_ids.astype(jnp.int32),
                    ((0, 0), (0, seq_p - seq))).reshape(-1)
    proj = _make_project(v, d, 1024)(emb_table, W, b.reshape(1, d))
    flat = _make_gather(n_seq * seq_p, d, 128)(proj, ids_p)
    return flat.reshape(n_seq, seq_p, d)[:, :seq, :]


# speedup vs baseline: 8.6342x; 8.6342x over previous
import functools

import jax
import jax.numpy as jnp
from jax import lax
from jax.experimental import pallas as pl
from jax.experimental.pallas import tpu as pltpu
from jax.experimental.pallas import tpu_sc as plsc

_NC = 2
_NS = 16
_NW = _NC * _NS


def _proj_body(x_ref, w_ref, b_ref, o_ref):
    o_ref[...] = lax.dot_general(
        x_ref[...], w_ref[...],
        dimension_numbers=(((1,), (1,)), ((), ())),
        preferred_element_type=jnp.float32,
    ) + b_ref[...]


@functools.lru_cache(maxsize=None)
def _make_project(v, d, bm):
    grid = (pl.cdiv(v, bm),)
    return pl.pallas_call(
        _proj_body,
        grid=grid,
        in_specs=[
            pl.BlockSpec((bm, d), lambda i: (i, 0)),
            pl.BlockSpec((d, d), lambda i: (0, 0)),
            pl.BlockSpec((1, d), lambda i: (0, 0)),
        ],
        out_specs=pl.BlockSpec((bm, d), lambda i: (i, 0)),
        out_shape=jax.ShapeDtypeStruct((v, d), jnp.float32),
    )


@functools.lru_cache(maxsize=None)
def _make_gather(b_total, d, ch):
    b_per_w = b_total // _NW
    n_ch = b_per_w // ch
    mesh = plsc.VectorSubcoreMesh(core_axis_name="c", subcore_axis_name="s")

    nbuf = 4
    assert n_ch % nbuf == 0

    @functools.partial(
        pl.kernel,
        mesh=mesh,
        out_type=jax.ShapeDtypeStruct((b_total, d), jnp.float32),
        scratch_types=[
            pltpu.VMEM((b_per_w,), jnp.int32),
        ] + [pltpu.VMEM((ch, d), jnp.float32)] * nbuf
          + [pltpu.SemaphoreType.DMA] * (2 * nbuf),
    )
    def gather_kernel(table_hbm, idx_hbm, out_hbm, idx_all, *bufs):
        rows = bufs[:nbuf]
        gsem = bufs[nbuf:2 * nbuf]
        wsem = bufs[2 * nbuf:]
        wid = lax.axis_index("s") * _NC + lax.axis_index("c")
        base = wid * b_per_w
        pltpu.sync_copy(idx_hbm.at[pl.ds(base, b_per_w)], idx_all)

        def g_start(slot, i):
            pltpu.async_copy(
                table_hbm.at[idx_all.at[pl.ds(i * ch, ch)]],
                rows[slot], gsem[slot])

        def g_wait(slot):
            pltpu.make_async_copy(table_hbm.at[pl.ds(0, ch)], rows[slot],
                                  gsem[slot]).wait()

        def w_start(slot, i):
            pltpu.async_copy(rows[slot], out_hbm.at[pl.ds(base + i * ch, ch)],
                             wsem[slot])

        def w_wait(slot):
            pltpu.make_async_copy(rows[slot], out_hbm.at[pl.ds(base, ch)],
                                  wsem[slot]).wait()

        for slot in range(nbuf):
            g_start(slot, slot)

        def outer(g, carry):
            for slot in range(nbuf):
                i = nbuf * g + slot
                g_wait(slot)
                w_start(slot, i)

                @pl.when(i + nbuf < n_ch)
                def _():
                    w_wait(slot)
                    g_start(slot, i + nbuf)
            return carry

        lax.fori_loop(0, n_ch // nbuf, outer, 0)
        for slot in range(nbuf):
            w_wait(slot)

    return gather_kernel


def kernel(input_ids, emb_table, W, b):
    v, d = emb_table.shape
    n_seq, seq = input_ids.shape
    idx_t = input_ids.astype(jnp.int32).T.reshape(-1)
    proj = _make_project(v, d, 1024)(emb_table, W, b.reshape(1, d))
    flat = _make_gather(seq * n_seq, d, 40)(proj, idx_t)
    return flat.reshape(seq, n_seq, d).transpose(1, 0, 2)
